# baseline (device time: 64762 ns/iter reference)
import jax
import jax.numpy as jnp
from jax import lax
from jax.experimental import pallas as pl
from jax.experimental.pallas import tpu as pltpu

N_DEV = 32
M_PER = 256
K = 8192
N_TOT = 4096
N_PER = 128
GRP = 2
GW = GRP * N_PER
N_GRP = N_TOT // GW


def _send_desc(y_buf, rbuf, send_sems, recv_sems, slot, j, my, dst):
    return pltpu.make_async_remote_copy(
        src_ref=y_buf.at[slot, :, pl.ds(j * N_PER, N_PER)],
        dst_ref=rbuf.at[pl.ds(my * M_PER, M_PER), :],
        send_sem=send_sems.at[dst],
        recv_sem=recv_sems.at[my],
        device_id=(dst,),
        device_id_type=pl.DeviceIdType.MESH,
    )


def _body(sched_ref, x_ref, w_ref, out_ref, y_buf, rbuf, send_sems, recv_sems):
    t = pl.program_id(0)
    my = lax.axis_index("i")
    g = sched_ref[t]
    slot = lax.rem(t, 4)

    barrier_sem = pltpu.get_barrier_semaphore()

    @pl.when(t == 0)
    def _():
        for p in range(N_DEV):
            @pl.when(p != my)
            def _():
                pl.semaphore_signal(
                    barrier_sem, inc=1,
                    device_id=(p,), device_id_type=pl.DeviceIdType.MESH,
                )

    @pl.when(t >= 4)
    def _():
        g_old = sched_ref[t - 4]
        for j in range(GRP):
            dst = g_old * GRP + j

            @pl.when(dst != my)
            def _():
                _send_desc(y_buf, rbuf, send_sems, recv_sems, slot, j, my, dst
                           ).wait_send()

    y_buf[slot] = jnp.maximum(
        jnp.dot(x_ref[...], w_ref[...], preferred_element_type=jnp.float32),
        0.0,
    ).astype(jnp.bfloat16)

    @pl.when(t == 0)
    def _():
        pl.semaphore_wait(barrier_sem, N_DEV - 1)

    for j in range(GRP):
        dst = g * GRP + j

        @pl.when(dst != my)
        def _():
            _send_desc(y_buf, rbuf, send_sems, recv_sems, slot, j, my, dst
                       ).start()

        @pl.when(dst == my)
        def _():
            rbuf[pl.ds(my * M_PER, M_PER), :] = y_buf[
                slot, :, pl.ds(j * N_PER, N_PER)
            ]

    @pl.when(t == N_GRP - 1)
    def _():
        d_seq = [8, 1, 9, 2, 10, 3, 11, 4, 12, 5, 13, 6, 14, 7, 15, 16]
        q = my // GRP
        for tt in range(N_GRP):
            ps = lax.rem(q + 2 * N_GRP - d_seq[tt], N_GRP)
            for j in range(GRP):
                src = ps * GRP + j

                @pl.when(src != my)
                def _():
                    pltpu.make_async_remote_copy(
                        src_ref=y_buf.at[0, :, pl.ds(0, N_PER)],
                        dst_ref=rbuf.at[pl.ds(src * M_PER, M_PER), :],
                        send_sem=send_sems.at[src],
                        recv_sem=recv_sems.at[src],
                        device_id=(src,),
                        device_id_type=pl.DeviceIdType.MESH,
                    ).wait_recv()
            rows = GRP * M_PER
            out_ref[pl.ds(ps * rows, rows), :] = rbuf[
                pl.ds(ps * rows, rows), :
            ].astype(jnp.float32)

        for tt in range(N_GRP - 4, N_GRP):
            g_late = sched_ref[tt]
            for j in range(GRP):
                dst = g_late * GRP + j

                @pl.when(dst != my)
                def _():
                    _send_desc(y_buf, rbuf, send_sems, recv_sems,
                               lax.rem(tt, 4), j, my, dst).wait_send()


def kernel(x, w_mat):
    my = lax.axis_index("i")
    d_seq = jnp.array([8, 1, 9, 2, 10, 3, 11, 4, 12, 5, 13, 6, 14, 7, 15, 16],
                      dtype=jnp.int32)
    sched = lax.rem(my // GRP + d_seq, N_GRP)

    grid_spec = pltpu.PrefetchScalarGridSpec(
        num_scalar_prefetch=1,
        grid=(N_GRP,),
        in_specs=[
            pl.BlockSpec((M_PER, K), lambda t, s: (0, 0)),
            pl.BlockSpec((K, GW), lambda t, s: (0, s[t])),
        ],
        out_specs=pl.BlockSpec((N_DEV * M_PER, N_PER), lambda t, s: (0, 0)),
        scratch_shapes=[
            pltpu.VMEM((4, M_PER, GW), jnp.bfloat16),
            pltpu.VMEM((N_DEV * M_PER, N_PER), jnp.bfloat16),
            pltpu.SemaphoreType.DMA((N_DEV,)),
            pltpu.SemaphoreType.DMA((N_DEV,)),
        ],
    )
    return pl.pallas_call(
        _body,
        grid_spec=grid_spec,
        out_shape=jax.ShapeDtypeStruct((N_DEV * M_PER, N_PER), jnp.float32),
        compiler_params=pltpu.CompilerParams(
            vmem_limit_bytes=60 * 1024 * 1024, collective_id=0,
        ),
    )(sched, x, w_mat)


# device time: 63764 ns/iter; 1.0157x vs baseline; 1.0157x over previous
import jax
import jax.numpy as jnp
from jax import lax
from jax.experimental import pallas as pl
from jax.experimental.pallas import tpu as pltpu

N_DEV = 32
M_PER = 256
K = 8192
N_TOT = 4096
N_PER = 128
GRP = 2
GW = GRP * N_PER
N_GRP = N_TOT // GW


def _send_desc(y_buf, rbuf, send_sems, recv_sems, slot, j, my, dst):
    return pltpu.make_async_remote_copy(
        src_ref=y_buf.at[slot, :, pl.ds(j * N_PER, N_PER)],
        dst_ref=rbuf.at[pl.ds(my * M_PER, M_PER), :],
        send_sem=send_sems.at[dst],
        recv_sem=recv_sems.at[my],
        device_id=(dst,),
        device_id_type=pl.DeviceIdType.MESH,
    )


def _body(sched_ref, x_ref, w_ref, out_ref, y_buf, rbuf, send_sems, recv_sems):
    t = pl.program_id(0)
    my = lax.axis_index("i")
    g = sched_ref[t]
    slot = lax.rem(t, 4)

    barrier_sem = pltpu.get_barrier_semaphore()

    @pl.when(t == 0)
    def _():
        for p in range(N_DEV):
            @pl.when(p != my)
            def _():
                pl.semaphore_signal(
                    barrier_sem, inc=1,
                    device_id=(p,), device_id_type=pl.DeviceIdType.MESH,
                )

    @pl.when(t >= 4)
    def _():
        g_old = sched_ref[t - 4]
        for j in range(GRP):
            dst = g_old * GRP + j

            @pl.when(dst != my)
            def _():
                _send_desc(y_buf, rbuf, send_sems, recv_sems, slot, j, my, dst
                           ).wait_send()

    y_buf[slot] = jnp.maximum(
        jnp.dot(x_ref[...], w_ref[...], preferred_element_type=jnp.float32),
        0.0,
    ).astype(jnp.bfloat16)

    @pl.when(t == 0)
    def _():
        pl.semaphore_wait(barrier_sem, N_DEV - 1)

    for j in range(GRP):
        dst = g * GRP + j

        @pl.when(dst != my)
        def _():
            _send_desc(y_buf, rbuf, send_sems, recv_sems, slot, j, my, dst
                       ).start()

        @pl.when(dst == my)
        def _():
            rbuf[pl.ds(my * M_PER, M_PER), :] = y_buf[
                slot, :, pl.ds(j * N_PER, N_PER)
            ]

    @pl.when(t == N_GRP - 1)
    def _():
        q = my // GRP
        for tt in range(N_GRP):
            ps = lax.rem(q + 2 * N_GRP - 1 - tt, N_GRP)
            for j in range(GRP):
                src = ps * GRP + j

                @pl.when(src != my)
                def _():
                    pltpu.make_async_remote_copy(
                        src_ref=y_buf.at[0, :, pl.ds(0, N_PER)],
                        dst_ref=rbuf.at[pl.ds(src * M_PER, M_PER), :],
                        send_sem=send_sems.at[src],
                        recv_sem=recv_sems.at[src],
                        device_id=(src,),
                        device_id_type=pl.DeviceIdType.MESH,
                    ).wait_recv()
            rows = GRP * M_PER
            out_ref[pl.ds(ps * rows, rows), :] = rbuf[
                pl.ds(ps * rows, rows), :
            ].astype(jnp.float32)

        for tt in range(N_GRP - 4, N_GRP):
            g_late = sched_ref[tt]
            for j in range(GRP):
                dst = g_late * GRP + j

                @pl.when(dst != my)
                def _():
                    _send_desc(y_buf, rbuf, send_sems, recv_sems,
                               lax.rem(tt, 4), j, my, dst).wait_send()


def kernel(x, w_mat):
    my = lax.axis_index("i")
    sched = lax.rem(my // GRP + 1 + jnp.arange(N_GRP, dtype=jnp.int32), N_GRP)

    grid_spec = pltpu.PrefetchScalarGridSpec(
        num_scalar_prefetch=1,
        grid=(N_GRP,),
        in_specs=[
            pl.BlockSpec((M_PER, K), lambda t, s: (0, 0)),
            pl.BlockSpec((K, GW), lambda t, s: (0, s[t])),
        ],
        out_specs=pl.BlockSpec((N_DEV * M_PER, N_PER), lambda t, s: (0, 0)),
        scratch_shapes=[
            pltpu.VMEM((4, M_PER, GW), jnp.bfloat16),
            pltpu.VMEM((N_DEV * M_PER, N_PER), jnp.bfloat16),
            pltpu.SemaphoreType.DMA((N_DEV,)),
            pltpu.SemaphoreType.DMA((N_DEV,)),
        ],
    )
    return pl.pallas_call(
        _body,
        grid_spec=grid_spec,
        out_shape=jax.ShapeDtypeStruct((N_DEV * M_PER, N_PER), jnp.float32),
        compiler_params=pltpu.CompilerParams(
            vmem_limit_bytes=60 * 1024 * 1024, collective_id=0,
        ),
    )(sched, x, w_mat)
